# baseline (speedup 1.0000x reference)
import functools

import jax
import jax.numpy as jnp
from jax import lax
from jax.experimental import pallas as pl
from jax.experimental.pallas import tpu as pltpu
from jax.experimental.pallas import tpu_sc as plsc

NP = 50000
NQ = 50000
E = 800000
H = 64
A = 5
NEG_SLOPE = 0.2

CH = 128
NSUB = 16
NPAD = 50048
RPT = NPAD // NSUB
HALF_CHUNKS = (E // 2) // CH
HALF_PER_TILE = -(-HALF_CHUNKS // NSUB)
FULL_CHUNKS = E // CH
FULL_PER_TILE = -(-FULL_CHUNKS // NSUB)


def _leaky(x):
    return jnp.where(x >= 0, x, NEG_SLOPE * x)


def _mesh():
    return plsc.VectorSubcoreMesh(core_axis_name="c", subcore_axis_name="s")


NCHUNK = 6272
HCHUNK = NCHUNK // 2
GD = 4
G2 = 2
NG = 49
NG2 = 196


def _sc_deg_body(dst_pp, dst_qq, ones_hbm, zeros_hbm, out_pp, out_qq,
                 ones_v, idx_v, acc, sem_i):
    c = lax.axis_index("c")
    s = lax.axis_index("s")
    r0 = s * RPT
    pltpu.sync_copy(ones_hbm, ones_v)

    rb = c * HCHUNK + s * (HCHUNK // NSUB)

    for dst2d, out in ((dst_pp, out_pp), (dst_qq, out_qq)):
        pltpu.sync_copy(zeros_hbm, acc.at[pl.ds(r0, RPT)])
        plsc.subcore_barrier()
        pltpu.async_copy(dst2d.at[pl.ds(rb, GD)], idx_v.at[0], sem_i.at[0])
        pltpu.async_copy(dst2d.at[pl.ds(rb + GD, GD)], idx_v.at[1], sem_i.at[1])

        def emit(g, b, prefetch):
            row0 = rb + g * GD
            pltpu.make_async_copy(dst2d.at[pl.ds(row0, GD)], idx_v.at[b],
                                  sem_i.at[b]).wait()
            for j in range(GD):
                pltpu.sync_copy(ones_v, acc.at[idx_v.at[b, j]], add=True)
            if prefetch:
                @pl.when(g + 2 < NG)
                def _():
                    pltpu.async_copy(dst2d.at[pl.ds(row0 + 2 * GD, GD)],
                                     idx_v.at[b], sem_i.at[b])

        def pair(k, carry):
            g = 2 * k
            emit(g, 0, True)
            emit(g + 1, 1, True)
            return carry

        lax.fori_loop(0, NG // 2, pair, 0)
        if NG % 2:
            emit(NG - 1, (NG - 1) % 2, False)
        plsc.subcore_barrier()
        pltpu.sync_copy(acc.at[pl.ds(r0, RPT)], out.at[c, pl.ds(r0, RPT)])
        plsc.subcore_barrier()


@functools.cache
def _sc_deg():
    return pl.kernel(
        _sc_deg_body,
        mesh=_mesh(),
        compiler_params=pltpu.CompilerParams(use_tc_tiling_on_sc=False),
        out_type=[
            jax.ShapeDtypeStruct((2, NPAD, 8), jnp.float32),
            jax.ShapeDtypeStruct((2, NPAD, 8), jnp.float32),
        ],
        scratch_types=[
            pltpu.VMEM((CH, 8), jnp.float32),
            pltpu.VMEM((2, GD, CH), jnp.int32),
            pltpu.VMEM_SHARED((NPAD, 8), jnp.float32),
            pltpu.SemaphoreType.DMA((2,)),
        ],
    )


def _agg_rel(tab, src2d, dst2d, acc, rb, G, ng,
             idx_s, idx_d, rows, sem_is, sem_g, sem_sc):
    pltpu.async_copy(src2d.at[pl.ds(rb, G)], idx_s.at[0], sem_is.at[0])
    pltpu.async_copy(src2d.at[pl.ds(rb + G, G)], idx_s.at[1], sem_is.at[1])

    def emit(g, b, prefetch):
        row0 = rb + g * G
        pltpu.make_async_copy(src2d.at[pl.ds(row0, G)], idx_s.at[b],
                              sem_is.at[b]).wait()
        for j in range(G):
            pltpu.async_copy(tab.at[idx_s.at[b, j]], rows.at[b, j],
                             sem_g.at[b])
        pltpu.sync_copy(dst2d.at[pl.ds(row0, G)], idx_d.at[b])
        for j in range(G):
            pltpu.make_async_copy(tab.at[idx_s.at[b, j]], rows.at[b, j],
                                  sem_g.at[b]).wait()
        if prefetch:
            @pl.when(g + 2 < ng)
            def _():
                pltpu.async_copy(src2d.at[pl.ds(row0 + 2 * G, G)],
                                 idx_s.at[b], sem_is.at[b])
        for j in range(G):
            pltpu.sync_copy(rows.at[b, j], acc.at[idx_d.at[b, j]], add=True)

    def pair(k, carry):
        g = 2 * k
        emit(g, 0, True)
        emit(g + 1, 1, True)
        return carry

    lax.fori_loop(0, ng // 2, pair, 0)
    if ng % 2:
        emit(ng - 1, (ng - 1) % 2, False)


def _sc_agg8_body(tab_pp, tab_qp, tab_qq,
                  src_pp, dst_pp, src_qp, dst_qp, src_qq, dst_qq, zeros_hbm,
                  out_pp, out_qp, out_qq,
                  idx_s, idx_d, rows, acc,
                  sem_is, sem_g, sem_sc):
    c = lax.axis_index("c")
    s = lax.axis_index("s")
    r0 = s * RPT
    rb = c * HCHUNK + s * (HCHUNK // NSUB)

    for tab, src2d, dst2d, out in ((tab_pp, src_pp, dst_pp, out_pp),
                                   (tab_qp, src_qp, dst_qp, out_qp),
                                   (tab_qq, src_qq, dst_qq, out_qq)):
        pltpu.sync_copy(zeros_hbm, acc.at[pl.ds(r0, RPT)])
        plsc.subcore_barrier()
        _agg_rel(tab, src2d, dst2d, acc, rb, GD, NG,
                 idx_s, idx_d, rows, sem_is, sem_g, sem_sc)
        plsc.subcore_barrier()
        pltpu.sync_copy(acc.at[pl.ds(r0, RPT)], out.at[c, pl.ds(r0, RPT)])
        plsc.subcore_barrier()


@functools.cache
def _sc_agg8():
    return pl.kernel(
        _sc_agg8_body,
        mesh=_mesh(),
        compiler_params=pltpu.CompilerParams(use_tc_tiling_on_sc=False),
        out_type=[
            jax.ShapeDtypeStruct((2, NPAD, 8), jnp.float32),
            jax.ShapeDtypeStruct((2, NPAD, 8), jnp.float32),
            jax.ShapeDtypeStruct((2, NPAD, 8), jnp.float32),
        ],
        scratch_types=[
            pltpu.VMEM((2, GD, CH), jnp.int32),
            pltpu.VMEM((2, GD, CH), jnp.int32),
            pltpu.VMEM((2, GD, CH, 8), jnp.float32),
            pltpu.VMEM_SHARED((NPAD, 8), jnp.float32),
            pltpu.SemaphoreType.DMA((2,)),
            pltpu.SemaphoreType.DMA((2,)),
            pltpu.SemaphoreType.DMA((2,)),
        ],
    )


def _sc_agg32_body(tab_p2, tab_q, src_pp, dst_pp, src_qp, dst_qp, zeros_hbm,
                   out_pp, out_qp,
                   idx_s, idx_d, rows, acc, sem_is, sem_g, sem_sc):
    c = lax.axis_index("c")
    s = lax.axis_index("s")
    r0 = s * RPT
    rb = s * (NCHUNK // NSUB)

    for tab, src2d, dst2d, out in ((tab_p2, src_pp, dst_pp, out_pp),
                                   (tab_q, src_qp, dst_qp, out_qp)):
        pltpu.sync_copy(zeros_hbm, acc.at[pl.ds(r0, RPT)])
        plsc.subcore_barrier()
        _agg_rel(tab.at[c], src2d, dst2d, acc, rb, G2, NG2,
                 idx_s, idx_d, rows, sem_is, sem_g, sem_sc)
        plsc.subcore_barrier()
        pltpu.sync_copy(acc.at[pl.ds(r0, RPT)], out.at[c, pl.ds(r0, RPT)])
        plsc.subcore_barrier()


@functools.cache
def _sc_agg32():
    return pl.kernel(
        _sc_agg32_body,
        mesh=_mesh(),
        compiler_params=pltpu.CompilerParams(use_tc_tiling_on_sc=False),
        out_type=[
            jax.ShapeDtypeStruct((2, NPAD, 32), jnp.float32),
            jax.ShapeDtypeStruct((2, NPAD, 32), jnp.float32),
        ],
        scratch_types=[
            pltpu.VMEM((2, G2, CH), jnp.int32),
            pltpu.VMEM((2, G2, CH), jnp.int32),
            pltpu.VMEM((2, G2, CH, 32), jnp.float32),
            pltpu.VMEM_SHARED((NPAD, 32), jnp.float32),
            pltpu.SemaphoreType.DMA((2,)),
            pltpu.SemaphoreType.DMA((2,)),
            pltpu.SemaphoreType.DMA((2,)),
        ],
    )


BLK = 2000
GRID = NP // BLK


def _tc_prep1(xp_ref, xq_ref, dpp_ref, dqq_ref, ypp_ref, xaug_ref, yqq_ref):
    z4 = jnp.zeros((BLK, 4), jnp.float32)
    deg_p = dpp_ref[0, :, 0] + dpp_ref[1, :, 0]
    dis_p = jnp.where(deg_p > 0, lax.rsqrt(jnp.maximum(deg_p, 1e-12)), 0.0)
    ypp_ref[...] = jnp.concatenate(
        [xp_ref[...] * dis_p[:, None], dis_p[:, None], z4], axis=1)
    deg_q = dqq_ref[0, :, 0] + dqq_ref[1, :, 0]
    dis_q = jnp.where(deg_q > 0, lax.rsqrt(jnp.maximum(deg_q, 1e-12)), 0.0)
    yqq_ref[...] = jnp.concatenate(
        [xq_ref[...] * dis_q[:, None], dis_q[:, None], z4], axis=1)
    xaug_ref[...] = jnp.concatenate(
        [xq_ref[...], jnp.ones((BLK, 1), jnp.float32), z4], axis=1)


def _tc_layer1(app_ref, aqp_ref, aqq_ref, ypp_ref, yqq_ref, xp_ref,
               Wpp1_ref, bpp1_ref, Wl1_ref, bl1_ref, Wr1_ref, Wqq1_ref, bqq1_ref,
               p_ref, tabp2_ref, tabq_ref, aux_ref):
    dis_p = ypp_ref[:, 3]
    app = app_ref[0] + app_ref[1]
    gcn = (app[:, :3] * dis_p[:, None]) @ Wpp1_ref[...] + bpp1_ref[...]
    aqp = aqp_ref[0] + aqp_ref[1]
    inv = 1.0 / jnp.maximum(aqp[:, 3], 1.0)
    sage = (aqp[:, :3] * inv[:, None]) @ Wl1_ref[...] + bl1_ref[...] \
        + xp_ref[...] @ Wr1_ref[...]
    p = _leaky(gcn + sage)
    dis_q = yqq_ref[:, 3]
    aqq = aqq_ref[0] + aqq_ref[1]
    q = _leaky((aqq[:, :3] * dis_q[:, None]) @ Wqq1_ref[...] + bqq1_ref[...])
    p_ref[...] = p
    y2 = p * dis_p[:, None]
    tabp2_ref[0] = y2[:, :32]
    tabp2_ref[1] = y2[:, 32:]
    tabq_ref[0] = q[:, :32]
    tabq_ref[1] = q[:, 32:]
    aux_ref[...] = jnp.concatenate(
        [dis_p[:, None], inv[:, None], jnp.zeros((BLK, 6), jnp.float32)], axis=1)


def _tc_layer2(a2pp_ref, a2qp_ref, p_ref, aux_ref,
               Wpp2_ref, bpp2_ref, Wl2_ref, bl2_ref, Wr2_ref, Wpost_ref, bpost_ref,
               out_ref):
    dis_p = aux_ref[:, 0]
    inv = aux_ref[:, 1]
    a2 = jnp.concatenate([a2pp_ref[0], a2pp_ref[1]], axis=1)
    gcn = (a2 * dis_p[:, None]) @ Wpp2_ref[...] + bpp2_ref[...]
    aq = jnp.concatenate([a2qp_ref[0], a2qp_ref[1]], axis=1)
    sage = (aq * inv[:, None]) @ Wl2_ref[...] + bl2_ref[...] \
        + p_ref[...] @ Wr2_ref[...]
    p2 = _leaky(gcn + sage)
    logits = p2 @ Wpost_ref[...] + bpost_ref[...]
    m = jnp.max(logits, axis=-1, keepdims=True)
    lse = jnp.log(jnp.sum(jnp.exp(logits - m), axis=-1, keepdims=True))
    out_ref[...] = logits - m - lse


def _row_spec(width):
    return pl.BlockSpec((BLK, width), lambda i: (i, 0))


def _part_spec(width):
    return pl.BlockSpec((2, BLK, width), lambda i: (0, i, 0))


def _w_spec(shape):
    nd = len(shape)
    return pl.BlockSpec(shape, lambda i: (0,) * nd)


def kernel(x_player, x_pellet, ei_pp, ei_qp, ei_qq,
           Wpp1, bpp1, Wl1, bl1, Wr1, Wqq1, bqq1,
           Wpp2, bpp2, Wl2, bl2, Wr2, Wqq2, bqq2,
           Wpost, bpost):
    f32 = jnp.float32
    i32 = jnp.int32
    ones8 = jnp.ones((CH, 8), f32)
    zeros8 = jnp.zeros((RPT, 8), f32)
    zeros32 = jnp.zeros((RPT, 32), f32)
    npad_e = NCHUNK * CH - E
    pad_s = jnp.zeros((npad_e,), i32)
    pad_d = jnp.full((npad_e,), NPAD - 1, i32)
    def _chunks(v, pad):
        return jnp.concatenate([v, pad]).reshape(NCHUNK, CH)
    src_pp, dst_pp = _chunks(ei_pp[0], pad_s), _chunks(ei_pp[1], pad_d)
    src_qp, dst_qp = _chunks(ei_qp[0], pad_s), _chunks(ei_qp[1], pad_d)
    src_qq, dst_qq = _chunks(ei_qq[0], pad_s), _chunks(ei_qq[1], pad_d)

    deg_pp, deg_qq = _sc_deg()(dst_pp, dst_qq, ones8, zeros8)

    ypp, xaug, yqq = pl.pallas_call(
        _tc_prep1,
        grid=(GRID,),
        in_specs=[_row_spec(3), _row_spec(3), _part_spec(8), _part_spec(8)],
        out_specs=[_row_spec(8), _row_spec(8), _row_spec(8)],
        out_shape=[jax.ShapeDtypeStruct((NP, 8), f32),
                   jax.ShapeDtypeStruct((NQ, 8), f32),
                   jax.ShapeDtypeStruct((NQ, 8), f32)],
    )(x_player, x_pellet, deg_pp, deg_qq)

    agg_pp1, agg_qp1, agg_qq1 = _sc_agg8()(
        ypp, xaug, yqq, src_pp, dst_pp, src_qp, dst_qp, src_qq, dst_qq, zeros8)

    b = lambda v: v.reshape(1, -1)
    p, tab_p2, tab_q, aux = pl.pallas_call(
        _tc_layer1,
        grid=(GRID,),
        in_specs=[_part_spec(8), _part_spec(8), _part_spec(8),
                  _row_spec(8), _row_spec(8), _row_spec(3),
                  _w_spec((3, H)), _w_spec((1, H)), _w_spec((3, H)),
                  _w_spec((1, H)), _w_spec((3, H)), _w_spec((3, H)),
                  _w_spec((1, H))],
        out_specs=[_row_spec(H), _part_spec(32), _part_spec(32), _row_spec(8)],
        out_shape=[jax.ShapeDtypeStruct((NP, H), f32),
                   jax.ShapeDtypeStruct((2, NP, 32), f32),
                   jax.ShapeDtypeStruct((2, NQ, 32), f32),
                   jax.ShapeDtypeStruct((NP, 8), f32)],
    )(agg_pp1, agg_qp1, agg_qq1, ypp, yqq, x_player,
      Wpp1, b(bpp1), Wl1, b(bl1), Wr1, Wqq1, b(bqq1))

    agg2_pp, agg2_qp = _sc_agg32()(tab_p2, tab_q, src_pp, dst_pp,
                                   src_qp, dst_qp, zeros32)

    out = pl.pallas_call(
        _tc_layer2,
        grid=(GRID,),
        in_specs=[_part_spec(32), _part_spec(32), _row_spec(H), _row_spec(8),
                  _w_spec((H, H)), _w_spec((1, H)), _w_spec((H, H)),
                  _w_spec((1, H)), _w_spec((H, H)), _w_spec((H, A)),
                  _w_spec((1, A))],
        out_specs=_row_spec(A),
        out_shape=jax.ShapeDtypeStruct((NP, A), f32),
    )(agg2_pp, agg2_qp, p, aux,
      Wpp2, b(bpp2), Wl2, b(bl2), Wr2, Wpost, b(bpost))

    return out


# speedup vs baseline: 1.1350x; 1.1350x over previous
import functools

import jax
import jax.numpy as jnp
from jax import lax
from jax.experimental import pallas as pl
from jax.experimental.pallas import tpu as pltpu
from jax.experimental.pallas import tpu_sc as plsc

NP = 50000
NQ = 50000
E = 800000
H = 64
A = 5
NEG_SLOPE = 0.2

CH = 128
NSUB = 16
NPAD = 50048
RPT = NPAD // NSUB
HALF_CHUNKS = (E // 2) // CH
HALF_PER_TILE = -(-HALF_CHUNKS // NSUB)
FULL_CHUNKS = E // CH
FULL_PER_TILE = -(-FULL_CHUNKS // NSUB)


def _leaky(x):
    return jnp.where(x >= 0, x, NEG_SLOPE * x)


def _mesh():
    return plsc.VectorSubcoreMesh(core_axis_name="c", subcore_axis_name="s")


NCHUNK = 6272
HCHUNK = NCHUNK // 2
GD = 4
G2 = 2
NG = 49
NG2 = 196


def _sc_deg_body(dst_pp, dst_qq, ones_hbm, zeros_hbm, out_pp, out_qq,
                 ones_v, idx_v, acc, sem_i):
    c = lax.axis_index("c")
    s = lax.axis_index("s")
    r0 = s * RPT
    pltpu.sync_copy(ones_hbm, ones_v)

    rb = c * HCHUNK + s * (HCHUNK // NSUB)

    for dst2d, out in ((dst_pp, out_pp), (dst_qq, out_qq)):
        pltpu.sync_copy(zeros_hbm, acc.at[pl.ds(r0, RPT)])
        plsc.subcore_barrier()
        pltpu.async_copy(dst2d.at[pl.ds(rb, GD)], idx_v.at[0], sem_i.at[0])
        pltpu.async_copy(dst2d.at[pl.ds(rb + GD, GD)], idx_v.at[1], sem_i.at[1])

        def emit(g, b, prefetch):
            row0 = rb + g * GD
            pltpu.make_async_copy(dst2d.at[pl.ds(row0, GD)], idx_v.at[b],
                                  sem_i.at[b]).wait()
            for j in range(GD):
                pltpu.sync_copy(ones_v, acc.at[idx_v.at[b, j]], add=True)
            if prefetch:
                @pl.when(g + 2 < NG)
                def _():
                    pltpu.async_copy(dst2d.at[pl.ds(row0 + 2 * GD, GD)],
                                     idx_v.at[b], sem_i.at[b])

        def pair(k, carry):
            g = 2 * k
            emit(g, 0, True)
            emit(g + 1, 1, True)
            return carry

        lax.fori_loop(0, NG // 2, pair, 0)
        if NG % 2:
            emit(NG - 1, (NG - 1) % 2, False)
        plsc.subcore_barrier()
        pltpu.sync_copy(acc.at[pl.ds(r0, RPT)], out.at[c, pl.ds(r0, RPT)])
        plsc.subcore_barrier()


@functools.cache
def _sc_deg():
    return pl.kernel(
        _sc_deg_body,
        mesh=_mesh(),
        compiler_params=pltpu.CompilerParams(use_tc_tiling_on_sc=False),
        out_type=[
            jax.ShapeDtypeStruct((2, NPAD, 8), jnp.float32),
            jax.ShapeDtypeStruct((2, NPAD, 8), jnp.float32),
        ],
        scratch_types=[
            pltpu.VMEM((CH, 8), jnp.float32),
            pltpu.VMEM((2, GD, CH), jnp.int32),
            pltpu.VMEM_SHARED((NPAD, 8), jnp.float32),
            pltpu.SemaphoreType.DMA((2,)),
        ],
    )


def _agg_rel(tab, src2d, dst2d, acc, rb, G, ng,
             idx_s, idx_d, rows, sem_is, sem_g, sem_sc):
    pltpu.async_copy(src2d.at[pl.ds(rb, G)], idx_s.at[0], sem_is.at[0])
    pltpu.async_copy(src2d.at[pl.ds(rb + G, G)], idx_s.at[1], sem_is.at[1])

    def emit(g, b, prefetch):
        row0 = rb + g * G

        @pl.when(g >= 2)
        def _():
            for j in range(G):
                pltpu.make_async_copy(rows.at[b, j], acc.at[idx_d.at[b, j]],
                                      sem_sc.at[b]).wait()

        pltpu.make_async_copy(src2d.at[pl.ds(row0, G)], idx_s.at[b],
                              sem_is.at[b]).wait()
        for j in range(G):
            pltpu.async_copy(tab.at[idx_s.at[b, j]], rows.at[b, j],
                             sem_g.at[b])
        pltpu.sync_copy(dst2d.at[pl.ds(row0, G)], idx_d.at[b])
        for j in range(G):
            pltpu.make_async_copy(tab.at[idx_s.at[b, j]], rows.at[b, j],
                                  sem_g.at[b]).wait()
        if prefetch:
            @pl.when(g + 2 < ng)
            def _():
                pltpu.async_copy(src2d.at[pl.ds(row0 + 2 * G, G)],
                                 idx_s.at[b], sem_is.at[b])
        for j in range(G):
            pltpu.async_copy(rows.at[b, j], acc.at[idx_d.at[b, j]],
                             sem_sc.at[b], add=True)

    def pair(k, carry):
        g = 2 * k
        emit(g, 0, True)
        emit(g + 1, 1, True)
        return carry

    lax.fori_loop(0, ng // 2, pair, 0)
    if ng % 2:
        emit(ng - 1, (ng - 1) % 2, False)
    for g in (ng - 2, ng - 1):
        b = g % 2
        for j in range(G):
            pltpu.make_async_copy(rows.at[b, j], acc.at[idx_d.at[b, j]],
                                  sem_sc.at[b]).wait()


def _sc_agg8_body(tab_pp, tab_qp, tab_qq,
                  src_pp, dst_pp, src_qp, dst_qp, src_qq, dst_qq, zeros_hbm,
                  out_pp, out_qp, out_qq,
                  idx_s, idx_d, rows, acc,
                  sem_is, sem_g, sem_sc):
    c = lax.axis_index("c")
    s = lax.axis_index("s")
    r0 = s * RPT
    rb = c * HCHUNK + s * (HCHUNK // NSUB)

    for tab, src2d, dst2d, out in ((tab_pp, src_pp, dst_pp, out_pp),
                                   (tab_qp, src_qp, dst_qp, out_qp),
                                   (tab_qq, src_qq, dst_qq, out_qq)):
        pltpu.sync_copy(zeros_hbm, acc.at[pl.ds(r0, RPT)])
        plsc.subcore_barrier()
        _agg_rel(tab, src2d, dst2d, acc, rb, GD, NG,
                 idx_s, idx_d, rows, sem_is, sem_g, sem_sc)
        plsc.subcore_barrier()
        pltpu.sync_copy(acc.at[pl.ds(r0, RPT)], out.at[c, pl.ds(r0, RPT)])
        plsc.subcore_barrier()


@functools.cache
def _sc_agg8():
    return pl.kernel(
        _sc_agg8_body,
        mesh=_mesh(),
        compiler_params=pltpu.CompilerParams(use_tc_tiling_on_sc=False),
        out_type=[
            jax.ShapeDtypeStruct((2, NPAD, 8), jnp.float32),
            jax.ShapeDtypeStruct((2, NPAD, 8), jnp.float32),
            jax.ShapeDtypeStruct((2, NPAD, 8), jnp.float32),
        ],
        scratch_types=[
            pltpu.VMEM((2, GD, CH), jnp.int32),
            pltpu.VMEM((2, GD, CH), jnp.int32),
            pltpu.VMEM((2, GD, CH, 8), jnp.float32),
            pltpu.VMEM_SHARED((NPAD, 8), jnp.float32),
            pltpu.SemaphoreType.DMA((2,)),
            pltpu.SemaphoreType.DMA((2,)),
            pltpu.SemaphoreType.DMA((2,)),
        ],
    )


def _sc_agg32_body(tab_p2, tab_q, src_pp, dst_pp, src_qp, dst_qp, zeros_hbm,
                   out_pp, out_qp,
                   idx_s, idx_d, rows, acc, sem_is, sem_g, sem_sc):
    c = lax.axis_index("c")
    s = lax.axis_index("s")
    r0 = s * RPT
    rb = s * (NCHUNK // NSUB)

    for tab, src2d, dst2d, out in ((tab_p2, src_pp, dst_pp, out_pp),
                                   (tab_q, src_qp, dst_qp, out_qp)):
        pltpu.sync_copy(zeros_hbm, acc.at[pl.ds(r0, RPT)])
        plsc.subcore_barrier()
        _agg_rel(tab.at[c], src2d, dst2d, acc, rb, G2, NG2,
                 idx_s, idx_d, rows, sem_is, sem_g, sem_sc)
        plsc.subcore_barrier()
        pltpu.sync_copy(acc.at[pl.ds(r0, RPT)], out.at[c, pl.ds(r0, RPT)])
        plsc.subcore_barrier()


@functools.cache
def _sc_agg32():
    return pl.kernel(
        _sc_agg32_body,
        mesh=_mesh(),
        compiler_params=pltpu.CompilerParams(use_tc_tiling_on_sc=False),
        out_type=[
            jax.ShapeDtypeStruct((2, NPAD, 32), jnp.float32),
            jax.ShapeDtypeStruct((2, NPAD, 32), jnp.float32),
        ],
        scratch_types=[
            pltpu.VMEM((2, G2, CH), jnp.int32),
            pltpu.VMEM((2, G2, CH), jnp.int32),
            pltpu.VMEM((2, G2, CH, 32), jnp.float32),
            pltpu.VMEM_SHARED((NPAD, 32), jnp.float32),
            pltpu.SemaphoreType.DMA((2,)),
            pltpu.SemaphoreType.DMA((2,)),
            pltpu.SemaphoreType.DMA((2,)),
        ],
    )


BLK = 2000
GRID = NP // BLK


def _tc_prep1(xp_ref, xq_ref, dpp_ref, dqq_ref, ypp_ref, xaug_ref, yqq_ref):
    z4 = jnp.zeros((BLK, 4), jnp.float32)
    deg_p = dpp_ref[0, :, 0] + dpp_ref[1, :, 0]
    dis_p = jnp.where(deg_p > 0, lax.rsqrt(jnp.maximum(deg_p, 1e-12)), 0.0)
    ypp_ref[...] = jnp.concatenate(
        [xp_ref[...] * dis_p[:, None], dis_p[:, None], z4], axis=1)
    deg_q = dqq_ref[0, :, 0] + dqq_ref[1, :, 0]
    dis_q = jnp.where(deg_q > 0, lax.rsqrt(jnp.maximum(deg_q, 1e-12)), 0.0)
    yqq_ref[...] = jnp.concatenate(
        [xq_ref[...] * dis_q[:, None], dis_q[:, None], z4], axis=1)
    xaug_ref[...] = jnp.concatenate(
        [xq_ref[...], jnp.ones((BLK, 1), jnp.float32), z4], axis=1)


def _tc_layer1(app_ref, aqp_ref, aqq_ref, ypp_ref, yqq_ref, xp_ref,
               Wpp1_ref, bpp1_ref, Wl1_ref, bl1_ref, Wr1_ref, Wqq1_ref, bqq1_ref,
               p_ref, tabp2_ref, tabq_ref, aux_ref):
    dis_p = ypp_ref[:, 3]
    app = app_ref[0] + app_ref[1]
    gcn = (app[:, :3] * dis_p[:, None]) @ Wpp1_ref[...] + bpp1_ref[...]
    aqp = aqp_ref[0] + aqp_ref[1]
    inv = 1.0 / jnp.maximum(aqp[:, 3], 1.0)
    sage = (aqp[:, :3] * inv[:, None]) @ Wl1_ref[...] + bl1_ref[...] \
        + xp_ref[...] @ Wr1_ref[...]
    p = _leaky(gcn + sage)
    dis_q = yqq_ref[:, 3]
    aqq = aqq_ref[0] + aqq_ref[1]
    q = _leaky((aqq[:, :3] * dis_q[:, None]) @ Wqq1_ref[...] + bqq1_ref[...])
    p_ref[...] = p
    y2 = p * dis_p[:, None]
    tabp2_ref[0] = y2[:, :32]
    tabp2_ref[1] = y2[:, 32:]
    tabq_ref[0] = q[:, :32]
    tabq_ref[1] = q[:, 32:]
    aux_ref[...] = jnp.concatenate(
        [dis_p[:, None], inv[:, None], jnp.zeros((BLK, 6), jnp.float32)], axis=1)


def _tc_layer2(a2pp_ref, a2qp_ref, p_ref, aux_ref,
               Wpp2_ref, bpp2_ref, Wl2_ref, bl2_ref, Wr2_ref, Wpost_ref, bpost_ref,
               out_ref):
    dis_p = aux_ref[:, 0]
    inv = aux_ref[:, 1]
    a2 = jnp.concatenate([a2pp_ref[0], a2pp_ref[1]], axis=1)
    gcn = (a2 * dis_p[:, None]) @ Wpp2_ref[...] + bpp2_ref[...]
    aq = jnp.concatenate([a2qp_ref[0], a2qp_ref[1]], axis=1)
    sage = (aq * inv[:, None]) @ Wl2_ref[...] + bl2_ref[...] \
        + p_ref[...] @ Wr2_ref[...]
    p2 = _leaky(gcn + sage)
    logits = p2 @ Wpost_ref[...] + bpost_ref[...]
    m = jnp.max(logits, axis=-1, keepdims=True)
    lse = jnp.log(jnp.sum(jnp.exp(logits - m), axis=-1, keepdims=True))
    out_ref[...] = logits - m - lse


def _row_spec(width):
    return pl.BlockSpec((BLK, width), lambda i: (i, 0))


def _part_spec(width):
    return pl.BlockSpec((2, BLK, width), lambda i: (0, i, 0))


def _w_spec(shape):
    nd = len(shape)
    return pl.BlockSpec(shape, lambda i: (0,) * nd)


def kernel(x_player, x_pellet, ei_pp, ei_qp, ei_qq,
           Wpp1, bpp1, Wl1, bl1, Wr1, Wqq1, bqq1,
           Wpp2, bpp2, Wl2, bl2, Wr2, Wqq2, bqq2,
           Wpost, bpost):
    f32 = jnp.float32
    i32 = jnp.int32
    ones8 = jnp.ones((CH, 8), f32)
    zeros8 = jnp.zeros((RPT, 8), f32)
    zeros32 = jnp.zeros((RPT, 32), f32)
    npad_e = NCHUNK * CH - E
    pad_s = jnp.zeros((npad_e,), i32)
    pad_d = jnp.full((npad_e,), NPAD - 1, i32)
    def _chunks(v, pad):
        return jnp.concatenate([v, pad]).reshape(NCHUNK, CH)
    src_pp, dst_pp = _chunks(ei_pp[0], pad_s), _chunks(ei_pp[1], pad_d)
    src_qp, dst_qp = _chunks(ei_qp[0], pad_s), _chunks(ei_qp[1], pad_d)
    src_qq, dst_qq = _chunks(ei_qq[0], pad_s), _chunks(ei_qq[1], pad_d)

    deg_pp, deg_qq = _sc_deg()(dst_pp, dst_qq, ones8, zeros8)

    ypp, xaug, yqq = pl.pallas_call(
        _tc_prep1,
        grid=(GRID,),
        in_specs=[_row_spec(3), _row_spec(3), _part_spec(8), _part_spec(8)],
        out_specs=[_row_spec(8), _row_spec(8), _row_spec(8)],
        out_shape=[jax.ShapeDtypeStruct((NP, 8), f32),
                   jax.ShapeDtypeStruct((NQ, 8), f32),
                   jax.ShapeDtypeStruct((NQ, 8), f32)],
    )(x_player, x_pellet, deg_pp, deg_qq)

    agg_pp1, agg_qp1, agg_qq1 = _sc_agg8()(
        ypp, xaug, yqq, src_pp, dst_pp, src_qp, dst_qp, src_qq, dst_qq, zeros8)

    b = lambda v: v.reshape(1, -1)
    p, tab_p2, tab_q, aux = pl.pallas_call(
        _tc_layer1,
        grid=(GRID,),
        in_specs=[_part_spec(8), _part_spec(8), _part_spec(8),
                  _row_spec(8), _row_spec(8), _row_spec(3),
                  _w_spec((3, H)), _w_spec((1, H)), _w_spec((3, H)),
                  _w_spec((1, H)), _w_spec((3, H)), _w_spec((3, H)),
                  _w_spec((1, H))],
        out_specs=[_row_spec(H), _part_spec(32), _part_spec(32), _row_spec(8)],
        out_shape=[jax.ShapeDtypeStruct((NP, H), f32),
                   jax.ShapeDtypeStruct((2, NP, 32), f32),
                   jax.ShapeDtypeStruct((2, NQ, 32), f32),
                   jax.ShapeDtypeStruct((NP, 8), f32)],
    )(agg_pp1, agg_qp1, agg_qq1, ypp, yqq, x_player,
      Wpp1, b(bpp1), Wl1, b(bl1), Wr1, Wqq1, b(bqq1))

    agg2_pp, agg2_qp = _sc_agg32()(tab_p2, tab_q, src_pp, dst_pp,
                                   src_qp, dst_qp, zeros32)

    out = pl.pallas_call(
        _tc_layer2,
        grid=(GRID,),
        in_specs=[_part_spec(32), _part_spec(32), _row_spec(H), _row_spec(8),
                  _w_spec((H, H)), _w_spec((1, H)), _w_spec((H, H)),
                  _w_spec((1, H)), _w_spec((H, H)), _w_spec((H, A)),
                  _w_spec((1, A))],
        out_specs=_row_spec(A),
        out_shape=jax.ShapeDtypeStruct((NP, A), f32),
    )(agg2_pp, agg2_qp, p, aux,
      Wpp2, b(bpp2), Wl2, b(bl2), Wr2, Wpost, b(bpost))

    return out


# speedup vs baseline: 1.1585x; 1.0208x over previous
import functools

import jax
import jax.numpy as jnp
from jax import lax
from jax.experimental import pallas as pl
from jax.experimental.pallas import tpu as pltpu
from jax.experimental.pallas import tpu_sc as plsc

NP = 50000
NQ = 50000
E = 800000
H = 64
A = 5
NEG_SLOPE = 0.2

CH = 128
NSUB = 16
NPAD = 50048
RPT = NPAD // NSUB
HALF_CHUNKS = (E // 2) // CH
HALF_PER_TILE = -(-HALF_CHUNKS // NSUB)
FULL_CHUNKS = E // CH
FULL_PER_TILE = -(-FULL_CHUNKS // NSUB)


def _leaky(x):
    return jnp.where(x >= 0, x, NEG_SLOPE * x)


def _mesh():
    return plsc.VectorSubcoreMesh(core_axis_name="c", subcore_axis_name="s")


NCHUNK = 6272
HCHUNK = NCHUNK // 2
GD = 4
G2 = 2
NG = 49
GA = 7
NGA = 28
NG2 = 196


def _sc_deg_body(dst_pp, dst_qq, ones_hbm, zeros_hbm, out_pp, out_qq,
                 ones_v, idx_v, acc, sem_i):
    c = lax.axis_index("c")
    s = lax.axis_index("s")
    r0 = s * RPT
    pltpu.sync_copy(ones_hbm, ones_v)

    rb = c * HCHUNK + s * (HCHUNK // NSUB)

    for dst2d, out in ((dst_pp, out_pp), (dst_qq, out_qq)):
        pltpu.sync_copy(zeros_hbm, acc.at[pl.ds(r0, RPT)])
        plsc.subcore_barrier()
        pltpu.async_copy(dst2d.at[pl.ds(rb, GD)], idx_v.at[0], sem_i.at[0])
        pltpu.async_copy(dst2d.at[pl.ds(rb + GD, GD)], idx_v.at[1], sem_i.at[1])

        def emit(g, b, prefetch):
            row0 = rb + g * GD
            pltpu.make_async_copy(dst2d.at[pl.ds(row0, GD)], idx_v.at[b],
                                  sem_i.at[b]).wait()
            for j in range(GD):
                pltpu.sync_copy(ones_v, acc.at[idx_v.at[b, j]], add=True)
            if prefetch:
                @pl.when(g + 2 < NG)
                def _():
                    pltpu.async_copy(dst2d.at[pl.ds(row0 + 2 * GD, GD)],
                                     idx_v.at[b], sem_i.at[b])

        def pair(k, carry):
            g = 2 * k
            emit(g, 0, True)
            emit(g + 1, 1, True)
            return carry

        lax.fori_loop(0, NG // 2, pair, 0)
        if NG % 2:
            emit(NG - 1, (NG - 1) % 2, False)
        plsc.subcore_barrier()
        pltpu.sync_copy(acc.at[pl.ds(r0, RPT)], out.at[c, pl.ds(r0, RPT)])
        plsc.subcore_barrier()


@functools.cache
def _sc_deg():
    return pl.kernel(
        _sc_deg_body,
        mesh=_mesh(),
        compiler_params=pltpu.CompilerParams(use_tc_tiling_on_sc=False),
        out_type=[
            jax.ShapeDtypeStruct((2, NPAD, 8), jnp.float32),
            jax.ShapeDtypeStruct((2, NPAD, 8), jnp.float32),
        ],
        scratch_types=[
            pltpu.VMEM((CH, 8), jnp.float32),
            pltpu.VMEM((2, GD, CH), jnp.int32),
            pltpu.VMEM_SHARED((NPAD, 8), jnp.float32),
            pltpu.SemaphoreType.DMA((2,)),
        ],
    )


def _agg_rel(tab, src2d, dst2d, acc, rb, G, ng,
             idx_s, idx_d, rows, sem_is, sem_id, sem_g, sem_sc):
    pltpu.async_copy(src2d.at[pl.ds(rb, G)], idx_s.at[0], sem_is.at[0])
    pltpu.async_copy(src2d.at[pl.ds(rb + G, G)], idx_s.at[1], sem_is.at[1])
    pltpu.async_copy(dst2d.at[pl.ds(rb, G)], idx_d.at[0], sem_id.at[0])
    pltpu.async_copy(dst2d.at[pl.ds(rb + G, G)], idx_d.at[1], sem_id.at[1])

    def emit(g, b2, b4):
        row0 = rb + g * G

        @pl.when(g >= 2)
        def _():
            for j in range(G):
                pltpu.make_async_copy(rows.at[b2, j], acc.at[idx_d.at[(b4 + 2) % 4, j]],
                                      sem_sc.at[b2]).wait()

        pltpu.make_async_copy(src2d.at[pl.ds(row0, G)], idx_s.at[b2],
                              sem_is.at[b2]).wait()
        for j in range(G):
            pltpu.async_copy(tab.at[idx_s.at[b2, j]], rows.at[b2, j],
                             sem_g.at[b2])
        pltpu.make_async_copy(dst2d.at[pl.ds(row0, G)], idx_d.at[b4],
                              sem_id.at[b4]).wait()
        for j in range(G):
            pltpu.make_async_copy(tab.at[idx_s.at[b2, j]], rows.at[b2, j],
                                  sem_g.at[b2]).wait()

        @pl.when(g + 2 < ng)
        def _():
            pltpu.async_copy(src2d.at[pl.ds(row0 + 2 * G, G)], idx_s.at[b2],
                             sem_is.at[b2])
            pltpu.async_copy(dst2d.at[pl.ds(row0 + 2 * G, G)],
                             idx_d.at[(b4 + 2) % 4], sem_id.at[(b4 + 2) % 4])

        for j in range(G):
            pltpu.async_copy(rows.at[b2, j], acc.at[idx_d.at[b4, j]],
                             sem_sc.at[b2], add=True)

    def quad(k, carry):
        g = 4 * k
        for i in range(4):
            emit(g + i, i % 2, i)
        return carry

    lax.fori_loop(0, ng // 4, quad, 0)
    for g in (ng - 2, ng - 1):
        b2, b4 = g % 2, g % 4
        for j in range(G):
            pltpu.make_async_copy(rows.at[b2, j], acc.at[idx_d.at[b4, j]],
                                  sem_sc.at[b2]).wait()


def _sc_agg8_body(tab_pp, tab_qp, tab_qq,
                  src_pp, dst_pp, src_qp, dst_qp, src_qq, dst_qq, zeros_hbm,
                  out_pp, out_qp, out_qq,
                  idx_s, idx_d, rows, acc,
                  sem_is, sem_id, sem_g, sem_sc):
    c = lax.axis_index("c")
    s = lax.axis_index("s")
    r0 = s * RPT
    rb = c * HCHUNK + s * (HCHUNK // NSUB)

    for tab, src2d, dst2d, out in ((tab_pp, src_pp, dst_pp, out_pp),
                                   (tab_qp, src_qp, dst_qp, out_qp),
                                   (tab_qq, src_qq, dst_qq, out_qq)):
        pltpu.sync_copy(zeros_hbm, acc.at[pl.ds(r0, RPT)])
        plsc.subcore_barrier()
        _agg_rel(tab, src2d, dst2d, acc, rb, GA, NGA,
                 idx_s, idx_d, rows, sem_is, sem_id, sem_g, sem_sc)
        plsc.subcore_barrier()
        pltpu.sync_copy(acc.at[pl.ds(r0, RPT)], out.at[c, pl.ds(r0, RPT)])
        plsc.subcore_barrier()


@functools.cache
def _sc_agg8():
    return pl.kernel(
        _sc_agg8_body,
        mesh=_mesh(),
        compiler_params=pltpu.CompilerParams(use_tc_tiling_on_sc=False),
        out_type=[
            jax.ShapeDtypeStruct((2, NPAD, 8), jnp.float32),
            jax.ShapeDtypeStruct((2, NPAD, 8), jnp.float32),
            jax.ShapeDtypeStruct((2, NPAD, 8), jnp.float32),
        ],
        scratch_types=[
            pltpu.VMEM((2, GA, CH), jnp.int32),
            pltpu.VMEM((4, GA, CH), jnp.int32),
            pltpu.VMEM((2, GA, CH, 8), jnp.float32),
            pltpu.VMEM_SHARED((NPAD, 8), jnp.float32),
            pltpu.SemaphoreType.DMA((2,)),
            pltpu.SemaphoreType.DMA((4,)),
            pltpu.SemaphoreType.DMA((2,)),
            pltpu.SemaphoreType.DMA((2,)),
        ],
    )


def _sc_agg32_body(tab_p2, tab_q, src_pp, dst_pp, src_qp, dst_qp, zeros_hbm,
                   out_pp, out_qp,
                   idx_s, idx_d, rows, acc, sem_is, sem_id, sem_g, sem_sc):
    c = lax.axis_index("c")
    s = lax.axis_index("s")
    r0 = s * RPT
    rb = s * (NCHUNK // NSUB)

    for tab, src2d, dst2d, out in ((tab_p2, src_pp, dst_pp, out_pp),
                                   (tab_q, src_qp, dst_qp, out_qp)):
        pltpu.sync_copy(zeros_hbm, acc.at[pl.ds(r0, RPT)])
        plsc.subcore_barrier()
        _agg_rel(tab.at[c], src2d, dst2d, acc, rb, G2, NG2,
                 idx_s, idx_d, rows, sem_is, sem_id, sem_g, sem_sc)
        plsc.subcore_barrier()
        pltpu.sync_copy(acc.at[pl.ds(r0, RPT)], out.at[c, pl.ds(r0, RPT)])
        plsc.subcore_barrier()


@functools.cache
def _sc_agg32():
    return pl.kernel(
        _sc_agg32_body,
        mesh=_mesh(),
        compiler_params=pltpu.CompilerParams(use_tc_tiling_on_sc=False),
        out_type=[
            jax.ShapeDtypeStruct((2, NPAD, 32), jnp.float32),
            jax.ShapeDtypeStruct((2, NPAD, 32), jnp.float32),
        ],
        scratch_types=[
            pltpu.VMEM((2, G2, CH), jnp.int32),
            pltpu.VMEM((4, G2, CH), jnp.int32),
            pltpu.VMEM((2, G2, CH, 32), jnp.float32),
            pltpu.VMEM_SHARED((NPAD, 32), jnp.float32),
            pltpu.SemaphoreType.DMA((2,)),
            pltpu.SemaphoreType.DMA((4,)),
            pltpu.SemaphoreType.DMA((2,)),
            pltpu.SemaphoreType.DMA((2,)),
        ],
    )


BLK = 2000
GRID = NP // BLK


def _tc_prep1(xp_ref, xq_ref, dpp_ref, dqq_ref, ypp_ref, xaug_ref, yqq_ref):
    z4 = jnp.zeros((BLK, 4), jnp.float32)
    deg_p = dpp_ref[0, :, 0] + dpp_ref[1, :, 0]
    dis_p = jnp.where(deg_p > 0, lax.rsqrt(jnp.maximum(deg_p, 1e-12)), 0.0)
    ypp_ref[...] = jnp.concatenate(
        [xp_ref[...] * dis_p[:, None], dis_p[:, None], z4], axis=1)
    deg_q = dqq_ref[0, :, 0] + dqq_ref[1, :, 0]
    dis_q = jnp.where(deg_q > 0, lax.rsqrt(jnp.maximum(deg_q, 1e-12)), 0.0)
    yqq_ref[...] = jnp.concatenate(
        [xq_ref[...] * dis_q[:, None], dis_q[:, None], z4], axis=1)
    xaug_ref[...] = jnp.concatenate(
        [xq_ref[...], jnp.ones((BLK, 1), jnp.float32), z4], axis=1)


def _tc_layer1(app_ref, aqp_ref, aqq_ref, ypp_ref, yqq_ref, xp_ref,
               Wpp1_ref, bpp1_ref, Wl1_ref, bl1_ref, Wr1_ref, Wqq1_ref, bqq1_ref,
               p_ref, tabp2_ref, tabq_ref, aux_ref):
    dis_p = ypp_ref[:, 3]
    app = app_ref[0] + app_ref[1]
    gcn = (app[:, :3] * dis_p[:, None]) @ Wpp1_ref[...] + bpp1_ref[...]
    aqp = aqp_ref[0] + aqp_ref[1]
    inv = 1.0 / jnp.maximum(aqp[:, 3], 1.0)
    sage = (aqp[:, :3] * inv[:, None]) @ Wl1_ref[...] + bl1_ref[...] \
        + xp_ref[...] @ Wr1_ref[...]
    p = _leaky(gcn + sage)
    dis_q = yqq_ref[:, 3]
    aqq = aqq_ref[0] + aqq_ref[1]
    q = _leaky((aqq[:, :3] * dis_q[:, None]) @ Wqq1_ref[...] + bqq1_ref[...])
    p_ref[...] = p
    y2 = p * dis_p[:, None]
    tabp2_ref[0] = y2[:, :32]
    tabp2_ref[1] = y2[:, 32:]
    tabq_ref[0] = q[:, :32]
    tabq_ref[1] = q[:, 32:]
    aux_ref[...] = jnp.concatenate(
        [dis_p[:, None], inv[:, None], jnp.zeros((BLK, 6), jnp.float32)], axis=1)


def _tc_layer2(a2pp_ref, a2qp_ref, p_ref, aux_ref,
               Wpp2_ref, bpp2_ref, Wl2_ref, bl2_ref, Wr2_ref, Wpost_ref, bpost_ref,
               out_ref):
    dis_p = aux_ref[:, 0]
    inv = aux_ref[:, 1]
    a2 = jnp.concatenate([a2pp_ref[0], a2pp_ref[1]], axis=1)
    gcn = (a2 * dis_p[:, None]) @ Wpp2_ref[...] + bpp2_ref[...]
    aq = jnp.concatenate([a2qp_ref[0], a2qp_ref[1]], axis=1)
    sage = (aq * inv[:, None]) @ Wl2_ref[...] + bl2_ref[...] \
        + p_ref[...] @ Wr2_ref[...]
    p2 = _leaky(gcn + sage)
    logits = p2 @ Wpost_ref[...] + bpost_ref[...]
    m = jnp.max(logits, axis=-1, keepdims=True)
    lse = jnp.log(jnp.sum(jnp.exp(logits - m), axis=-1, keepdims=True))
    out_ref[...] = logits - m - lse


def _row_spec(width):
    return pl.BlockSpec((BLK, width), lambda i: (i, 0))


def _part_spec(width):
    return pl.BlockSpec((2, BLK, width), lambda i: (0, i, 0))


def _w_spec(shape):
    nd = len(shape)
    return pl.BlockSpec(shape, lambda i: (0,) * nd)


def kernel(x_player, x_pellet, ei_pp, ei_qp, ei_qq,
           Wpp1, bpp1, Wl1, bl1, Wr1, Wqq1, bqq1,
           Wpp2, bpp2, Wl2, bl2, Wr2, Wqq2, bqq2,
           Wpost, bpost):
    f32 = jnp.float32
    i32 = jnp.int32
    ones8 = jnp.ones((CH, 8), f32)
    zeros8 = jnp.zeros((RPT, 8), f32)
    zeros32 = jnp.zeros((RPT, 32), f32)
    npad_e = NCHUNK * CH - E
    pad_s = jnp.zeros((npad_e,), i32)
    pad_d = jnp.full((npad_e,), NPAD - 1, i32)
    def _chunks(v, pad):
        return jnp.concatenate([v, pad]).reshape(NCHUNK, CH)
    src_pp, dst_pp = _chunks(ei_pp[0], pad_s), _chunks(ei_pp[1], pad_d)
    src_qp, dst_qp = _chunks(ei_qp[0], pad_s), _chunks(ei_qp[1], pad_d)
    src_qq, dst_qq = _chunks(ei_qq[0], pad_s), _chunks(ei_qq[1], pad_d)

    deg_pp, deg_qq = _sc_deg()(dst_pp, dst_qq, ones8, zeros8)

    ypp, xaug, yqq = pl.pallas_call(
        _tc_prep1,
        grid=(GRID,),
        in_specs=[_row_spec(3), _row_spec(3), _part_spec(8), _part_spec(8)],
        out_specs=[_row_spec(8), _row_spec(8), _row_spec(8)],
        out_shape=[jax.ShapeDtypeStruct((NP, 8), f32),
                   jax.ShapeDtypeStruct((NQ, 8), f32),
                   jax.ShapeDtypeStruct((NQ, 8), f32)],
    )(x_player, x_pellet, deg_pp, deg_qq)

    agg_pp1, agg_qp1, agg_qq1 = _sc_agg8()(
        ypp, xaug, yqq, src_pp, dst_pp, src_qp, dst_qp, src_qq, dst_qq, zeros8)

    b = lambda v: v.reshape(1, -1)
    p, tab_p2, tab_q, aux = pl.pallas_call(
        _tc_layer1,
        grid=(GRID,),
        in_specs=[_part_spec(8), _part_spec(8), _part_spec(8),
                  _row_spec(8), _row_spec(8), _row_spec(3),
                  _w_spec((3, H)), _w_spec((1, H)), _w_spec((3, H)),
                  _w_spec((1, H)), _w_spec((3, H)), _w_spec((3, H)),
                  _w_spec((1, H))],
        out_specs=[_row_spec(H), _part_spec(32), _part_spec(32), _row_spec(8)],
        out_shape=[jax.ShapeDtypeStruct((NP, H), f32),
                   jax.ShapeDtypeStruct((2, NP, 32), f32),
                   jax.ShapeDtypeStruct((2, NQ, 32), f32),
                   jax.ShapeDtypeStruct((NP, 8), f32)],
    )(agg_pp1, agg_qp1, agg_qq1, ypp, yqq, x_player,
      Wpp1, b(bpp1), Wl1, b(bl1), Wr1, Wqq1, b(bqq1))

    agg2_pp, agg2_qp = _sc_agg32()(tab_p2, tab_q, src_pp, dst_pp,
                                   src_qp, dst_qp, zeros32)

    out = pl.pallas_call(
        _tc_layer2,
        grid=(GRID,),
        in_specs=[_part_spec(32), _part_spec(32), _row_spec(H), _row_spec(8),
                  _w_spec((H, H)), _w_spec((1, H)), _w_spec((H, H)),
                  _w_spec((1, H)), _w_spec((H, H)), _w_spec((H, A)),
                  _w_spec((1, A))],
        out_specs=_row_spec(A),
        out_shape=jax.ShapeDtypeStruct((NP, A), f32),
    )(agg2_pp, agg2_qp, p, aux,
      Wpp2, b(bpp2), Wl2, b(bl2), Wr2, Wpost, b(bpost))

    return out
